# baseline (speedup 1.0000x reference)
import jax
import jax.numpy as jnp
from jax import lax
from jax.experimental import pallas as pl
from jax.experimental.pallas import tpu as pltpu
from jax.experimental.pallas import tpu_sc as plsc

B = 4096
L = 200
VOCAB = 100000
EMB = 100
PK = 50
PK_P = 64
SUM_P = 128
HID = 64
OUT = 5

NC = 2
NS = 16
NW = NC * NS
ROWS_PER_W = B // NW
HALF = 100
CHUNKS_PER_W = 2 * ROWS_PER_W

WOFFS = (0, 16, 32, 34)
IOFFS = (0, 16, 32, 48, 64, 80, 84)


def _pool_body(x2_hbm, emb_hbm, out_hbm, idx_v, idx2_v, buf_v, acc_v,
               sem0, sem1, sem2, sem3):
    cid = lax.axis_index("c")
    sid = lax.axis_index("s")
    wid = sid * NC + cid
    base2 = wid * CHUNKS_PER_W
    baseb = wid * ROWS_PER_W

    pltpu.sync_copy(x2_hbm.at[pl.ds(base2, CHUNKS_PER_W)], idx_v)

    def tbody(c, carry):
        for off in IOFFS:
            v = idx_v[c, pl.ds(off, 16)]
            v2 = v + v
            idx2_v[c, pl.ds(off, 16)] = jnp.where(
                v < VOCAB // 2, v2, v2 - (VOCAB - 1))
        return carry

    lax.fori_loop(0, CHUNKS_PER_W, tbody, 0)

    sems = (sem0, sem1, sem2, sem3)
    NPAIR = 4

    def issue(b, p):
        c = 2 * b
        pltpu.async_copy(emb_hbm.at[idx2_v.at[c]], buf_v.at[2 * p], sems[p])
        pltpu.async_copy(emb_hbm.at[idx2_v.at[c + 1]], buf_v.at[2 * p + 1], sems[p])

    def wait(p):
        pltpu.make_async_copy(emb_hbm.at[idx2_v.at[0]], buf_v.at[2 * p], sems[p]).wait()
        pltpu.make_async_copy(emb_hbm.at[idx2_v.at[0]], buf_v.at[2 * p + 1], sems[p]).wait()

    issue(0, 0)
    issue(1, 1)

    def proc(b, p):
        @pl.when(b + 2 < ROWS_PER_W)
        def _():
            issue(b + 2, (p + 2) % NPAIR)

        wait(p)

        b0 = buf_v.at[2 * p]
        b1 = buf_v.at[2 * p + 1]

        def rbody(i, accs):
            r = 2 * i
            out = list(accs)
            for u in range(2):
                for half in (b0, b1):
                    for k, off in enumerate(WOFFS):
                        w = half[r + u, pl.ds(off, 16)]
                        pair = plsc.bitcast(w, jnp.bfloat16)
                        lo, hi = plsc.unpack(
                            pair, format=plsc.PackFormat.INTERLEAVED,
                            preferred_element_type=jnp.float32)
                        out[2 * k] = out[2 * k] + lo
                        out[2 * k + 1] = out[2 * k + 1] + hi
            return tuple(out)

        z = jnp.zeros((16,), jnp.float32)
        accs = lax.fori_loop(0, HALF // 2, rbody, (z,) * (2 * len(WOFFS)))
        for k, off in enumerate(WOFFS):
            acc_v[b, pl.ds(off, 16)] = accs[2 * k]
            acc_v[b, pl.ds(PK + off, 16)] = accs[2 * k + 1]

    def outer(bb, carry):
        for p in range(NPAIR):
            proc(NPAIR * bb + p, p)
        return carry

    lax.fori_loop(0, ROWS_PER_W // NPAIR, outer, 0)

    pltpu.sync_copy(acc_v, out_hbm.at[pl.ds(baseb, ROWS_PER_W)])


def _pool(x2, emb_pk):
    f = pl.kernel(
        _pool_body,
        out_type=jax.ShapeDtypeStruct((B, SUM_P), jnp.float32),
        mesh=plsc.VectorSubcoreMesh(core_axis_name="c", subcore_axis_name="s"),
        scratch_types=[
            pltpu.VMEM((CHUNKS_PER_W, HALF), jnp.int32),
            pltpu.VMEM((CHUNKS_PER_W, HALF), jnp.int32),
            pltpu.VMEM((2 * 4, HALF, PK_P), jnp.int32),
            pltpu.VMEM((ROWS_PER_W, SUM_P), jnp.float32),
            pltpu.SemaphoreType.DMA,
            pltpu.SemaphoreType.DMA,
            pltpu.SemaphoreType.DMA,
            pltpu.SemaphoreType.DMA,
        ],
        compiler_params=pltpu.CompilerParams(
            use_tc_tiling_on_sc=False, needs_layout_passes=False),
    )
    return f(x2, emb_pk)


def _pack_table(emb):
    u = lax.bitcast_convert_type(emb, jnp.uint32)
    bb = (u + 0x7FFF + ((u >> jnp.uint32(16)) & jnp.uint32(1))) >> jnp.uint32(16)
    w = lax.bitcast_convert_type(
        bb[:, 0:PK] | (bb[:, PK:EMB] << jnp.uint32(16)), jnp.int32)
    v2 = VOCAB // 2
    z = jnp.zeros((v2, PK_P - PK), jnp.int32)
    pairs = jnp.concatenate([w[:v2], z, w[v2:], z], axis=1)
    return pairs.reshape(VOCAB, PK_P)


def _mlp_body(s_ref, w1_ref, b1_ref, w2_ref, b2_ref, o_ref):
    h = s_ref[:, 0:EMB] * (1.0 / L)
    h = jnp.dot(h, w1_ref[...], preferred_element_type=jnp.float32) + b1_ref[...]
    h = jnp.maximum(h, 0.0)
    logits = jnp.dot(h, w2_ref[...], preferred_element_type=jnp.float32) + b2_ref[...]
    m = jnp.max(logits, axis=1, keepdims=True)
    e = jnp.exp(logits - m)
    o_ref[...] = e / jnp.sum(e, axis=1, keepdims=True)


def _mlp(sums, W1, b1, W2, b2):
    return pl.pallas_call(
        _mlp_body,
        out_shape=jax.ShapeDtypeStruct((B, OUT), jnp.float32),
    )(sums, W1, b1.reshape(1, HID), W2, b2.reshape(1, OUT))


def kernel(x, emb, W1, b1, W2, b2):
    x2 = x.reshape(B * 2, HALF)
    emb_pk = _pack_table(emb)
    sums = _pool(x2, emb_pk)
    return _mlp(sums, W1, b1, W2, b2)


# speedup vs baseline: 1.1465x; 1.1465x over previous
import jax
import jax.numpy as jnp
from jax import lax
from jax.experimental import pallas as pl
from jax.experimental.pallas import tpu as pltpu
from jax.experimental.pallas import tpu_sc as plsc

B = 4096
L = 200
VOCAB = 100000
EMB = 100
PK = 50
PK_P = 64
SUM_P = 128
HID = 64
OUT = 5

NC = 2
NS = 16
NW = NC * NS
ROWS_PER_W = B // NW
HALF = 100
CHUNKS_PER_W = 2 * ROWS_PER_W

WOFFS = (0, 16, 32, 34)
IOFFS = (0, 16, 32, 48, 64, 80, 84)


def _pool_body(x2_hbm, emb_hbm, out_hbm, idx_v, idx2_v, buf_v, acc_v,
               sem0, sem1, sem2, sem3):
    cid = lax.axis_index("c")
    sid = lax.axis_index("s")
    wid = sid * NC + cid
    base2 = wid * CHUNKS_PER_W
    baseb = wid * ROWS_PER_W

    pltpu.sync_copy(x2_hbm.at[pl.ds(base2, CHUNKS_PER_W)], idx_v)

    def tbody(c, carry):
        for off in IOFFS:
            v = idx_v[c, pl.ds(off, 16)]
            v2 = v + v
            idx2_v[c, pl.ds(off, 16)] = jnp.where(
                v < VOCAB // 2, v2, v2 - (VOCAB - 1))
        return carry

    lax.fori_loop(0, CHUNKS_PER_W, tbody, 0)

    sems = (sem0, sem1, sem2, sem3)
    NPAIR = 4

    def issue(b, p):
        c = 2 * b
        pltpu.async_copy(emb_hbm.at[idx2_v.at[c]], buf_v.at[2 * p], sems[p])
        pltpu.async_copy(emb_hbm.at[idx2_v.at[c + 1]], buf_v.at[2 * p + 1], sems[p])

    def wait(p):
        pltpu.make_async_copy(emb_hbm.at[idx2_v.at[0]], buf_v.at[2 * p], sems[p]).wait()
        pltpu.make_async_copy(emb_hbm.at[idx2_v.at[0]], buf_v.at[2 * p + 1], sems[p]).wait()

    issue(0, 0)
    issue(1, 1)

    def proc(b, p):
        @pl.when(b + 2 < ROWS_PER_W)
        def _():
            issue(b + 2, (p + 2) % NPAIR)

        wait(p)

        b0 = buf_v.at[2 * p]
        b1 = buf_v.at[2 * p + 1]

        def rbody(i, accs):
            r = 2 * i
            out = list(accs)
            for u in range(2):
                for half in (b0, b1):
                    for k, off in enumerate(WOFFS):
                        w = half[r + u, pl.ds(off, 16)]
                        pair = plsc.bitcast(w, jnp.bfloat16)
                        lo, hi = plsc.unpack(
                            pair, format=plsc.PackFormat.INTERLEAVED,
                            preferred_element_type=jnp.float32)
                        out[2 * k] = out[2 * k] + lo
                        out[2 * k + 1] = out[2 * k + 1] + hi
            return tuple(out)

        z = jnp.zeros((16,), jnp.float32)
        accs = lax.fori_loop(0, HALF // 2, rbody, (z,) * (2 * len(WOFFS)))
        for k, off in enumerate(WOFFS):
            acc_v[b, pl.ds(off, 16)] = accs[2 * k]
            acc_v[b, pl.ds(PK + off, 16)] = accs[2 * k + 1]

    def outer(bb, carry):
        for p in range(NPAIR):
            proc(NPAIR * bb + p, p)
        return carry

    lax.fori_loop(0, ROWS_PER_W // NPAIR, outer, 0)

    pltpu.sync_copy(acc_v, out_hbm.at[pl.ds(baseb, ROWS_PER_W)])


def _pool(x2, emb_pk):
    f = pl.kernel(
        _pool_body,
        out_type=jax.ShapeDtypeStruct((B, SUM_P), jnp.float32),
        mesh=plsc.VectorSubcoreMesh(core_axis_name="c", subcore_axis_name="s"),
        scratch_types=[
            pltpu.VMEM((CHUNKS_PER_W, HALF), jnp.int32),
            pltpu.VMEM((CHUNKS_PER_W, HALF), jnp.int32),
            pltpu.VMEM((2 * 4, HALF, PK_P), jnp.int32),
            pltpu.VMEM((ROWS_PER_W, SUM_P), jnp.float32),
            pltpu.SemaphoreType.DMA,
            pltpu.SemaphoreType.DMA,
            pltpu.SemaphoreType.DMA,
            pltpu.SemaphoreType.DMA,
        ],
        compiler_params=pltpu.CompilerParams(
            use_tc_tiling_on_sc=False, needs_layout_passes=False),
    )
    return f(x2, emb_pk)


def _pack_table(emb):
    eb = emb.astype(jnp.bfloat16)
    lo_u = lax.bitcast_convert_type(eb[:, 0:PK], jnp.uint16).astype(jnp.uint32)
    hi_u = lax.bitcast_convert_type(eb[:, PK:EMB], jnp.uint16).astype(jnp.uint32)
    w = lax.bitcast_convert_type(lo_u | (hi_u << jnp.uint32(16)), jnp.int32)
    v2 = VOCAB // 2
    z = jnp.zeros((v2, PK_P - PK), jnp.int32)
    pairs = jnp.concatenate([w[:v2], z, w[v2:], z], axis=1)
    return pairs.reshape(VOCAB, PK_P)


def _mlp_body(s_ref, w1_ref, b1_ref, w2_ref, b2_ref, o_ref):
    h = s_ref[:, 0:EMB] * (1.0 / L)
    h = jnp.dot(h, w1_ref[...], preferred_element_type=jnp.float32) + b1_ref[...]
    h = jnp.maximum(h, 0.0)
    logits = jnp.dot(h, w2_ref[...], preferred_element_type=jnp.float32) + b2_ref[...]
    m = jnp.max(logits, axis=1, keepdims=True)
    e = jnp.exp(logits - m)
    o_ref[...] = e / jnp.sum(e, axis=1, keepdims=True)


def _mlp(sums, W1, b1, W2, b2):
    return pl.pallas_call(
        _mlp_body,
        out_shape=jax.ShapeDtypeStruct((B, OUT), jnp.float32),
    )(sums, W1, b1.reshape(1, HID), W2, b2.reshape(1, OUT))


def kernel(x, emb, W1, b1, W2, b2):
    x2 = x.reshape(B * 2, HALF)
    emb_pk = _pack_table(emb)
    sums = _pool(x2, emb_pk)
    return _mlp(sums, W1, b1, W2, b2)


# speedup vs baseline: 1.1518x; 1.0047x over previous
import jax
import jax.numpy as jnp
from jax import lax
from jax.experimental import pallas as pl
from jax.experimental.pallas import tpu as pltpu
from jax.experimental.pallas import tpu_sc as plsc

B = 4096
L = 200
VOCAB = 100000
EMB = 100
PK = 50
PK_P = 64
SUM_P = 128
HID = 64
OUT = 5

NC = 2
NS = 16
NW = NC * NS
ROWS_PER_W = B // NW
HALF = 100
CHUNKS_PER_W = 2 * ROWS_PER_W

WOFFS = (0, 16, 32, 34)
IOFFS = (0, 16, 32, 48, 64, 80, 84)


def _pool_body(x2_hbm, emb_hbm, out_hbm, idx_v, idx2_v, buf_v, acc_v,
               sem0, sem1, sem2, sem3):
    cid = lax.axis_index("c")
    sid = lax.axis_index("s")
    wid = sid * NC + cid
    base2 = wid * CHUNKS_PER_W
    baseb = wid * ROWS_PER_W

    pltpu.sync_copy(x2_hbm.at[pl.ds(base2, CHUNKS_PER_W)], idx_v)

    def tbody(c, carry):
        for off in IOFFS:
            v = idx_v[c, pl.ds(off, 16)]
            v2 = v + v
            idx2_v[c, pl.ds(off, 16)] = jnp.where(
                v < VOCAB // 2, v2, v2 - (VOCAB - 1))
        return carry

    lax.fori_loop(0, CHUNKS_PER_W, tbody, 0)

    sems = (sem0, sem1, sem2, sem3)
    NPAIR = 4

    def issue(b, p):
        c = 2 * b
        pltpu.async_copy(emb_hbm.at[idx2_v.at[c]], buf_v.at[2 * p], sems[p])
        pltpu.async_copy(emb_hbm.at[idx2_v.at[c + 1]], buf_v.at[2 * p + 1], sems[p])

    def wait(p):
        pltpu.make_async_copy(emb_hbm.at[idx2_v.at[0]], buf_v.at[2 * p], sems[p]).wait()
        pltpu.make_async_copy(emb_hbm.at[idx2_v.at[0]], buf_v.at[2 * p + 1], sems[p]).wait()

    issue(0, 0)
    issue(1, 1)
    issue(2, 2)

    def proc(b, p):
        @pl.when(b + 3 < ROWS_PER_W)
        def _():
            issue(b + 3, (p + 3) % NPAIR)

        wait(p)

        b0 = buf_v.at[2 * p]
        b1 = buf_v.at[2 * p + 1]

        def rbody(i, accs):
            r = 2 * i
            out = list(accs)
            for u in range(2):
                for half in (b0, b1):
                    for k, off in enumerate(WOFFS):
                        w = half[r + u, pl.ds(off, 16)]
                        pair = plsc.bitcast(w, jnp.bfloat16)
                        lo, hi = plsc.unpack(
                            pair, format=plsc.PackFormat.INTERLEAVED,
                            preferred_element_type=jnp.float32)
                        out[2 * k] = out[2 * k] + lo
                        out[2 * k + 1] = out[2 * k + 1] + hi
            return tuple(out)

        z = jnp.zeros((16,), jnp.float32)
        accs = lax.fori_loop(0, HALF // 2, rbody, (z,) * (2 * len(WOFFS)))
        for k, off in enumerate(WOFFS):
            acc_v[b, pl.ds(off, 16)] = accs[2 * k]
            acc_v[b, pl.ds(PK + off, 16)] = accs[2 * k + 1]

    def outer(bb, carry):
        for p in range(NPAIR):
            proc(NPAIR * bb + p, p)
        return carry

    lax.fori_loop(0, ROWS_PER_W // NPAIR, outer, 0)

    pltpu.sync_copy(acc_v, out_hbm.at[pl.ds(baseb, ROWS_PER_W)])


def _pool(x2, emb_pk):
    f = pl.kernel(
        _pool_body,
        out_type=jax.ShapeDtypeStruct((B, SUM_P), jnp.float32),
        mesh=plsc.VectorSubcoreMesh(core_axis_name="c", subcore_axis_name="s"),
        scratch_types=[
            pltpu.VMEM((CHUNKS_PER_W, HALF), jnp.int32),
            pltpu.VMEM((CHUNKS_PER_W, HALF), jnp.int32),
            pltpu.VMEM((2 * 4, HALF, PK_P), jnp.int32),
            pltpu.VMEM((ROWS_PER_W, SUM_P), jnp.float32),
            pltpu.SemaphoreType.DMA,
            pltpu.SemaphoreType.DMA,
            pltpu.SemaphoreType.DMA,
            pltpu.SemaphoreType.DMA,
        ],
        compiler_params=pltpu.CompilerParams(
            use_tc_tiling_on_sc=False, needs_layout_passes=False),
    )
    return f(x2, emb_pk)


def _pack_table(emb):
    eb = emb.astype(jnp.bfloat16)
    lo_u = lax.bitcast_convert_type(eb[:, 0:PK], jnp.uint16).astype(jnp.uint32)
    hi_u = lax.bitcast_convert_type(eb[:, PK:EMB], jnp.uint16).astype(jnp.uint32)
    w = lax.bitcast_convert_type(lo_u | (hi_u << jnp.uint32(16)), jnp.int32)
    v2 = VOCAB // 2
    z = jnp.zeros((v2, PK_P - PK), jnp.int32)
    pairs = jnp.concatenate([w[:v2], z, w[v2:], z], axis=1)
    return pairs.reshape(VOCAB, PK_P)


def _mlp_body(s_ref, w1_ref, b1_ref, w2_ref, b2_ref, o_ref):
    h = s_ref[:, 0:EMB] * (1.0 / L)
    h = jnp.dot(h, w1_ref[...], preferred_element_type=jnp.float32) + b1_ref[...]
    h = jnp.maximum(h, 0.0)
    logits = jnp.dot(h, w2_ref[...], preferred_element_type=jnp.float32) + b2_ref[...]
    m = jnp.max(logits, axis=1, keepdims=True)
    e = jnp.exp(logits - m)
    o_ref[...] = e / jnp.sum(e, axis=1, keepdims=True)


def _mlp(sums, W1, b1, W2, b2):
    return pl.pallas_call(
        _mlp_body,
        out_shape=jax.ShapeDtypeStruct((B, OUT), jnp.float32),
    )(sums, W1, b1.reshape(1, HID), W2, b2.reshape(1, OUT))


def kernel(x, emb, W1, b1, W2, b2):
    x2 = x.reshape(B * 2, HALF)
    emb_pk = _pack_table(emb)
    sums = _pool(x2, emb_pk)
    return _mlp(sums, W1, b1, W2, b2)
